# baseline (speedup 1.0000x reference)
import functools

import jax
import jax.numpy as jnp
from jax import lax
from jax.experimental import pallas as pl
from jax.experimental.pallas import tpu as pltpu
from jax.experimental.pallas import tpu_sc as plsc

Q_SZ = 1024
N_SZ = 50000
D_SZ = 256
K_SZ = 16
NP = 51200
QB = 8
QBE = 256

def _knn_body(qp8_ref, sp8_ref, idx_ref, d2_scr):
    qp = qp8_ref[...]
    sp = sp8_ref[...]
    qn = jnp.sum(qp * qp, axis=1, keepdims=True)
    kn = jnp.sum(sp * sp, axis=0, keepdims=True)
    dot = lax.dot_general(qp, sp, (((1,), (0,)), ((), ())),
                          preferred_element_type=jnp.float32)
    d2_scr[...] = qn + kn - 2.0 * dot
    iota = lax.broadcasted_iota(jnp.int32, (QB, NP), 1)
    cols = []
    pend = []

    def mask_pend(x):
        for a in pend:
            x = jnp.where(iota == a, jnp.float32(jnp.inf), x)
        return x

    for r in range(K_SZ):
        md = mask_pend(d2_scr[...])
        m = jnp.min(md, axis=1, keepdims=True)
        cand = jnp.where(md == m, iota, jnp.int32(2**30))
        am = jnp.min(cand, axis=1, keepdims=True)
        cols.append(am)
        pend.append(am)
        if len(pend) == 4 and r != K_SZ - 1:
            d2_scr[...] = mask_pend(d2_scr[...])
            pend = []
    idx_ref[...] = jnp.concatenate(cols, axis=1)


def _knn_call(qp8, sp8):
    return pl.pallas_call(
        _knn_body,
        grid=(Q_SZ // QB,),
        in_specs=[
            pl.BlockSpec((QB, 8), lambda i: (i, 0)),
            pl.BlockSpec((8, NP), lambda i: (0, 0)),
        ],
        out_specs=pl.BlockSpec((QB, K_SZ), lambda i: (i, 0)),
        out_shape=jax.ShapeDtypeStruct((Q_SZ, K_SZ), jnp.int32),
        scratch_shapes=[pltpu.VMEM((QB, NP), jnp.float32)],
    )(qp8, sp8)


def _make_gather():
    NC, NS = 2, 16
    NW = NC * NS
    B = Q_SZ * K_SZ
    b_per_w = B // NW
    CH = 128
    mesh = plsc.VectorSubcoreMesh(core_axis_name="c", subcore_axis_name="s")

    @functools.partial(
        pl.kernel,
        mesh=mesh,
        out_type=jax.ShapeDtypeStruct((B, D_SZ), jnp.float32),
        scratch_types=[
            pltpu.VMEM((CH,), jnp.int32),
            pltpu.VMEM((CH, D_SZ), jnp.float32),
            pltpu.SemaphoreType.DMA,
        ],
    )
    def gather(table_hbm, idx_hbm, out_hbm, idx_v, rows_v, sem):
        c = lax.axis_index("c")
        s = lax.axis_index("s")
        base = (s * NC + c) * b_per_w
        for j in range(b_per_w // CH):
            off = base + j * CH
            pltpu.sync_copy(idx_hbm.at[pl.ds(off, CH)], idx_v)
            pltpu.async_copy(table_hbm.at[idx_v], rows_v, sem).wait()
            pltpu.sync_copy(rows_v, out_hbm.at[pl.ds(off, CH)])

    return gather


def _epi_body(x_ref, feat_ref, wqT_ref, wkT_ref, wb_ref, wvT_ref, woT_ref,
              g_ref, b_ref, out_ref):
    x = x_ref[...]
    q = lax.dot_general(x, wqT_ref[...], (((1,), (0,)), ((), ())),
                        preferred_element_type=jnp.float32)
    logits = lax.dot_general(q, wkT_ref[...], (((1,), (0,)), ((), ())),
                             preferred_element_type=jnp.float32)
    logits = logits + wb_ref[0:1, :]
    mx = jnp.max(logits, axis=1, keepdims=True)
    e = jnp.exp(logits - mx)
    p = e / jnp.sum(e, axis=1, keepdims=True)
    f = feat_ref[...]
    agg = jnp.sum(f * p[:, :, None], axis=1)
    v = lax.dot_general(agg, wvT_ref[...], (((1,), (0,)), ((), ())),
                        preferred_element_type=jnp.float32)
    h = q * v
    o = lax.dot_general(h, woT_ref[...], (((1,), (0,)), ((), ())),
                        preferred_element_type=jnp.float32) + x
    mu = jnp.mean(o, axis=1, keepdims=True)
    var = jnp.mean((o - mu) ** 2, axis=1, keepdims=True)
    out_ref[...] = (o - mu) * lax.rsqrt(var + 1e-5) * g_ref[0:1, :] + b_ref[0:1, :]


def _epi_call(query, feat3, wqT, wkT, wb8, wvT, woT, g8, b8):
    return pl.pallas_call(
        _epi_body,
        grid=(Q_SZ // QBE,),
        in_specs=[
            pl.BlockSpec((QBE, D_SZ), lambda i: (i, 0)),
            pl.BlockSpec((QBE, K_SZ, D_SZ), lambda i: (i, 0, 0)),
            pl.BlockSpec((D_SZ, D_SZ), lambda i: (0, 0)),
            pl.BlockSpec((D_SZ, K_SZ), lambda i: (0, 0)),
            pl.BlockSpec((8, K_SZ), lambda i: (0, 0)),
            pl.BlockSpec((D_SZ, D_SZ), lambda i: (0, 0)),
            pl.BlockSpec((D_SZ, D_SZ), lambda i: (0, 0)),
            pl.BlockSpec((8, D_SZ), lambda i: (0, 0)),
            pl.BlockSpec((8, D_SZ), lambda i: (0, 0)),
        ],
        out_specs=pl.BlockSpec((QBE, D_SZ), lambda i: (i, 0)),
        out_shape=jax.ShapeDtypeStruct((Q_SZ, D_SZ), jnp.float32),
    )(query, feat3, wqT, wkT, wb8, wvT, woT, g8, b8)


def kernel(query, query_pos, inst_feats, sp_coords, w_q, w_v, w_o, w_k, w_b,
           ln_g, ln_b):
    qp8 = jnp.pad(query_pos, ((0, 0), (0, 5)))
    spT = jnp.pad(sp_coords.T, ((0, 0), (0, NP - N_SZ)),
                  constant_values=1.0e6)
    sp8 = jnp.pad(spT, ((0, 5), (0, 0)))

    idx = _knn_call(qp8, sp8)

    feat = _make_gather()(inst_feats, idx.reshape(-1))
    feat3 = feat.reshape(Q_SZ, K_SZ, D_SZ)

    wb8 = jnp.broadcast_to(w_b.reshape(1, K_SZ), (8, K_SZ))
    g8 = jnp.broadcast_to(ln_g.reshape(1, D_SZ), (8, D_SZ))
    b8 = jnp.broadcast_to(ln_b.reshape(1, D_SZ), (8, D_SZ))
    return _epi_call(query, feat3, w_q.T, w_k.T, wb8, w_v.T, w_o.T, g8, b8)


# speedup vs baseline: 1.7876x; 1.7876x over previous
import functools

import jax
import jax.numpy as jnp
from jax import lax
from jax.experimental import pallas as pl
from jax.experimental.pallas import tpu as pltpu
from jax.experimental.pallas import tpu_sc as plsc

Q_SZ = 1024
N_SZ = 50000
D_SZ = 256
K_SZ = 16
NP = 51200
QB = 32
QBE = 256

def _knn_body(qp8_ref, sp8_ref, idx_ref, d2_scr, iota_scr):
    qp = qp8_ref[...]
    sp = sp8_ref[...]
    qn = jnp.sum(qp * qp, axis=1, keepdims=True)
    kn = jnp.sum(sp * sp, axis=0, keepdims=True)
    dot = lax.dot_general(qp, sp, (((1,), (0,)), ((), ())),
                          preferred_element_type=jnp.float32)
    d2_scr[...] = qn + kn - 2.0 * dot
    iota_scr[...] = lax.broadcasted_iota(jnp.int32, (QB, NP), 1)
    cols = []
    for r in range(K_SZ):
        d2 = d2_scr[...]
        iota = iota_scr[...]
        m = jnp.min(d2, axis=1, keepdims=True)
        cand = jnp.where(d2 == m, iota, jnp.int32(2**30))
        am = jnp.min(cand, axis=1, keepdims=True)
        cols.append(am)
        if r != K_SZ - 1:
            d2_scr[...] = jnp.where(cand == am, jnp.float32(jnp.inf), d2)
    idx_ref[...] = jnp.concatenate(cols, axis=1)


def _knn_call(qp8, sp8):
    return pl.pallas_call(
        _knn_body,
        grid=(Q_SZ // QB,),
        in_specs=[
            pl.BlockSpec((QB, 8), lambda i: (i, 0)),
            pl.BlockSpec((8, NP), lambda i: (0, 0)),
        ],
        out_specs=pl.BlockSpec((QB, K_SZ), lambda i: (i, 0)),
        out_shape=jax.ShapeDtypeStruct((Q_SZ, K_SZ), jnp.int32),
        scratch_shapes=[pltpu.VMEM((QB, NP), jnp.float32),
                        pltpu.VMEM((QB, NP), jnp.int32)],
    )(qp8, sp8)


def _make_gather():
    NC, NS = 2, 16
    NW = NC * NS
    B = Q_SZ * K_SZ
    b_per_w = B // NW
    CH = 128
    mesh = plsc.VectorSubcoreMesh(core_axis_name="c", subcore_axis_name="s")

    @functools.partial(
        pl.kernel,
        mesh=mesh,
        out_type=jax.ShapeDtypeStruct((B, D_SZ), jnp.float32),
        scratch_types=[
            pltpu.VMEM((CH,), jnp.int32),
            pltpu.VMEM((CH, D_SZ), jnp.float32),
            pltpu.SemaphoreType.DMA,
        ],
    )
    def gather(table_hbm, idx_hbm, out_hbm, idx_v, rows_v, sem):
        c = lax.axis_index("c")
        s = lax.axis_index("s")
        base = (s * NC + c) * b_per_w
        for j in range(b_per_w // CH):
            off = base + j * CH
            pltpu.sync_copy(idx_hbm.at[pl.ds(off, CH)], idx_v)
            pltpu.async_copy(table_hbm.at[idx_v], rows_v, sem).wait()
            pltpu.sync_copy(rows_v, out_hbm.at[pl.ds(off, CH)])

    return gather


def _epi_body(x_ref, feat_ref, wqT_ref, wkT_ref, wb_ref, wvT_ref, woT_ref,
              g_ref, b_ref, out_ref):
    x = x_ref[...]
    q = lax.dot_general(x, wqT_ref[...], (((1,), (0,)), ((), ())),
                        preferred_element_type=jnp.float32)
    logits = lax.dot_general(q, wkT_ref[...], (((1,), (0,)), ((), ())),
                             preferred_element_type=jnp.float32)
    logits = logits + wb_ref[0:1, :]
    mx = jnp.max(logits, axis=1, keepdims=True)
    e = jnp.exp(logits - mx)
    p = e / jnp.sum(e, axis=1, keepdims=True)
    f = feat_ref[...]
    agg = jnp.sum(f * p[:, :, None], axis=1)
    v = lax.dot_general(agg, wvT_ref[...], (((1,), (0,)), ((), ())),
                        preferred_element_type=jnp.float32)
    h = q * v
    o = lax.dot_general(h, woT_ref[...], (((1,), (0,)), ((), ())),
                        preferred_element_type=jnp.float32) + x
    mu = jnp.mean(o, axis=1, keepdims=True)
    var = jnp.mean((o - mu) ** 2, axis=1, keepdims=True)
    out_ref[...] = (o - mu) * lax.rsqrt(var + 1e-5) * g_ref[0:1, :] + b_ref[0:1, :]


def _epi_call(query, feat3, wqT, wkT, wb8, wvT, woT, g8, b8):
    return pl.pallas_call(
        _epi_body,
        grid=(Q_SZ // QBE,),
        in_specs=[
            pl.BlockSpec((QBE, D_SZ), lambda i: (i, 0)),
            pl.BlockSpec((QBE, K_SZ, D_SZ), lambda i: (i, 0, 0)),
            pl.BlockSpec((D_SZ, D_SZ), lambda i: (0, 0)),
            pl.BlockSpec((D_SZ, K_SZ), lambda i: (0, 0)),
            pl.BlockSpec((8, K_SZ), lambda i: (0, 0)),
            pl.BlockSpec((D_SZ, D_SZ), lambda i: (0, 0)),
            pl.BlockSpec((D_SZ, D_SZ), lambda i: (0, 0)),
            pl.BlockSpec((8, D_SZ), lambda i: (0, 0)),
            pl.BlockSpec((8, D_SZ), lambda i: (0, 0)),
        ],
        out_specs=pl.BlockSpec((QBE, D_SZ), lambda i: (i, 0)),
        out_shape=jax.ShapeDtypeStruct((Q_SZ, D_SZ), jnp.float32),
    )(query, feat3, wqT, wkT, wb8, wvT, woT, g8, b8)


def kernel(query, query_pos, inst_feats, sp_coords, w_q, w_v, w_o, w_k, w_b,
           ln_g, ln_b):
    qp8 = jnp.pad(query_pos, ((0, 0), (0, 5)))
    spT = jnp.pad(sp_coords.T, ((0, 0), (0, NP - N_SZ)),
                  constant_values=1.0e6)
    sp8 = jnp.pad(spT, ((0, 5), (0, 0)))

    idx = _knn_call(qp8, sp8)

    feat = _make_gather()(inst_feats, idx.reshape(-1))
    feat3 = feat.reshape(Q_SZ, K_SZ, D_SZ)

    wb8 = jnp.broadcast_to(w_b.reshape(1, K_SZ), (8, K_SZ))
    g8 = jnp.broadcast_to(ln_g.reshape(1, D_SZ), (8, D_SZ))
    b8 = jnp.broadcast_to(ln_b.reshape(1, D_SZ), (8, D_SZ))
    return _epi_call(query, feat3, w_q.T, w_k.T, wb8, w_v.T, w_o.T, g8, b8)


# speedup vs baseline: 1.7885x; 1.0005x over previous
import functools

import jax
import jax.numpy as jnp
from jax import lax
from jax.experimental import pallas as pl
from jax.experimental.pallas import tpu as pltpu
from jax.experimental.pallas import tpu_sc as plsc

Q_SZ = 1024
N_SZ = 50000
D_SZ = 256
K_SZ = 16
NP = 51200
QB = 32
QBE = 256

def _knn_body(qp8_ref, sp8_ref, idx_ref, d2_scr):
    qp = qp8_ref[...]
    sp = sp8_ref[...]
    qn = jnp.sum(qp * qp, axis=1, keepdims=True)
    kn = jnp.sum(sp * sp, axis=0, keepdims=True)
    dot = lax.dot_general(qp, sp, (((1,), (0,)), ((), ())),
                          preferred_element_type=jnp.float32)
    d2_scr[...] = qn + kn - 2.0 * dot
    iota = lax.broadcasted_iota(jnp.int32, (QB, NP), 1)
    cols = []
    for r in range(K_SZ):
        d2 = d2_scr[...]
        m = jnp.min(d2, axis=1, keepdims=True)
        cand = jnp.where(d2 == m, iota, jnp.int32(2**30))
        am = jnp.min(cand, axis=1, keepdims=True)
        cols.append(am)
        if r != K_SZ - 1:
            d2_scr[...] = jnp.where(cand == am, jnp.float32(jnp.inf), d2)
    idx_ref[...] = jnp.concatenate(cols, axis=1)


def _knn_call(qp8, sp8):
    return pl.pallas_call(
        _knn_body,
        grid=(Q_SZ // QB,),
        in_specs=[
            pl.BlockSpec((QB, 8), lambda i: (i, 0)),
            pl.BlockSpec((8, NP), lambda i: (0, 0)),
        ],
        out_specs=pl.BlockSpec((QB, K_SZ), lambda i: (i, 0)),
        out_shape=jax.ShapeDtypeStruct((Q_SZ, K_SZ), jnp.int32),
        scratch_shapes=[pltpu.VMEM((QB, NP), jnp.float32)],
    )(qp8, sp8)


def _make_gather():
    NC, NS = 2, 16
    NW = NC * NS
    B = Q_SZ * K_SZ
    b_per_w = B // NW
    CH = 128
    mesh = plsc.VectorSubcoreMesh(core_axis_name="c", subcore_axis_name="s")

    @functools.partial(
        pl.kernel,
        mesh=mesh,
        out_type=jax.ShapeDtypeStruct((B, D_SZ), jnp.float32),
        scratch_types=[
            pltpu.VMEM((CH,), jnp.int32),
            pltpu.VMEM((CH, D_SZ), jnp.float32),
            pltpu.SemaphoreType.DMA,
        ],
    )
    def gather(table_hbm, idx_hbm, out_hbm, idx_v, rows_v, sem):
        c = lax.axis_index("c")
        s = lax.axis_index("s")
        base = (s * NC + c) * b_per_w
        for j in range(b_per_w // CH):
            off = base + j * CH
            pltpu.sync_copy(idx_hbm.at[pl.ds(off, CH)], idx_v)
            pltpu.async_copy(table_hbm.at[idx_v], rows_v, sem).wait()
            pltpu.sync_copy(rows_v, out_hbm.at[pl.ds(off, CH)])

    return gather


def _epi_body(x_ref, feat_ref, wqT_ref, wkT_ref, wb_ref, wvT_ref, woT_ref,
              g_ref, b_ref, out_ref):
    x = x_ref[...]
    q = lax.dot_general(x, wqT_ref[...], (((1,), (0,)), ((), ())),
                        preferred_element_type=jnp.float32)
    logits = lax.dot_general(q, wkT_ref[...], (((1,), (0,)), ((), ())),
                             preferred_element_type=jnp.float32)
    logits = logits + wb_ref[0:1, :]
    mx = jnp.max(logits, axis=1, keepdims=True)
    e = jnp.exp(logits - mx)
    p = e / jnp.sum(e, axis=1, keepdims=True)
    f = feat_ref[...]
    agg = jnp.sum(f * p[:, :, None], axis=1)
    v = lax.dot_general(agg, wvT_ref[...], (((1,), (0,)), ((), ())),
                        preferred_element_type=jnp.float32)
    h = q * v
    o = lax.dot_general(h, woT_ref[...], (((1,), (0,)), ((), ())),
                        preferred_element_type=jnp.float32) + x
    mu = jnp.mean(o, axis=1, keepdims=True)
    var = jnp.mean((o - mu) ** 2, axis=1, keepdims=True)
    out_ref[...] = (o - mu) * lax.rsqrt(var + 1e-5) * g_ref[0:1, :] + b_ref[0:1, :]


def _epi_call(query, feat3, wqT, wkT, wb8, wvT, woT, g8, b8):
    return pl.pallas_call(
        _epi_body,
        grid=(Q_SZ // QBE,),
        in_specs=[
            pl.BlockSpec((QBE, D_SZ), lambda i: (i, 0)),
            pl.BlockSpec((QBE, K_SZ, D_SZ), lambda i: (i, 0, 0)),
            pl.BlockSpec((D_SZ, D_SZ), lambda i: (0, 0)),
            pl.BlockSpec((D_SZ, K_SZ), lambda i: (0, 0)),
            pl.BlockSpec((8, K_SZ), lambda i: (0, 0)),
            pl.BlockSpec((D_SZ, D_SZ), lambda i: (0, 0)),
            pl.BlockSpec((D_SZ, D_SZ), lambda i: (0, 0)),
            pl.BlockSpec((8, D_SZ), lambda i: (0, 0)),
            pl.BlockSpec((8, D_SZ), lambda i: (0, 0)),
        ],
        out_specs=pl.BlockSpec((QBE, D_SZ), lambda i: (i, 0)),
        out_shape=jax.ShapeDtypeStruct((Q_SZ, D_SZ), jnp.float32),
    )(query, feat3, wqT, wkT, wb8, wvT, woT, g8, b8)


def kernel(query, query_pos, inst_feats, sp_coords, w_q, w_v, w_o, w_k, w_b,
           ln_g, ln_b):
    qp8 = jnp.pad(query_pos, ((0, 0), (0, 5)))
    spT = jnp.pad(sp_coords.T, ((0, 0), (0, NP - N_SZ)),
                  constant_values=1.0e6)
    sp8 = jnp.pad(spT, ((0, 5), (0, 0)))

    idx = _knn_call(qp8, sp8)

    feat = _make_gather()(inst_feats, idx.reshape(-1))
    feat3 = feat.reshape(Q_SZ, K_SZ, D_SZ)

    wb8 = jnp.broadcast_to(w_b.reshape(1, K_SZ), (8, K_SZ))
    g8 = jnp.broadcast_to(ln_g.reshape(1, D_SZ), (8, D_SZ))
    b8 = jnp.broadcast_to(ln_b.reshape(1, D_SZ), (8, D_SZ))
    return _epi_call(query, feat3, w_q.T, w_k.T, wb8, w_v.T, w_o.T, g8, b8)


# speedup vs baseline: 2.3596x; 1.3193x over previous
import functools

import jax
import jax.numpy as jnp
from jax import lax
from jax.experimental import pallas as pl
from jax.experimental.pallas import tpu as pltpu
from jax.experimental.pallas import tpu_sc as plsc

Q_SZ = 1024
N_SZ = 50000
D_SZ = 256
K_SZ = 16
NP = 51200
QB = 32
QBE = 256

def _knn_body(qp8_ref, sp8_ref, idx_ref, d2_scr):
    qp = qp8_ref[...]
    sp = sp8_ref[...]
    qn = jnp.sum(qp * qp, axis=1, keepdims=True)
    kn = jnp.sum(sp * sp, axis=0, keepdims=True)
    dot = lax.dot_general(qp, sp, (((1,), (0,)), ((), ())),
                          preferred_element_type=jnp.float32)
    d2_scr[...] = qn + kn - 2.0 * dot

    W0 = NP // K_SZ
    giota = lax.broadcasted_iota(jnp.int32, (QB, W0), 1)
    vals = [d2_scr[:, j * W0:(j + 1) * W0] for j in range(K_SZ)]
    idxs = [giota + j * W0 for j in range(K_SZ)]

    def lexlt(a, ia, b, ib):
        return (a < b) | ((a == b) & (ia < ib))

    def ce(i, j):
        a, ia, b, ib = vals[i], idxs[i], vals[j], idxs[j]
        c = lexlt(a, ia, b, ib)
        vals[i] = jnp.where(c, a, b)
        vals[j] = jnp.where(c, b, a)
        idxs[i] = jnp.where(c, ia, ib)
        idxs[j] = jnp.where(c, ib, ia)

    pairs = []

    def oemerge(lo, n, r):
        step = r * 2
        if step < n:
            oemerge(lo, n, step)
            oemerge(lo + r, n, step)
            for i in range(lo + r, lo + n - r, step):
                pairs.append((i, i + r))
        else:
            pairs.append((lo, lo + r))

    def oems(lo, n):
        if n > 1:
            m = n // 2
            oems(lo, m)
            oems(lo + m, m)
            oemerge(lo, n, 1)

    oems(0, K_SZ)
    for i, j in pairs:
        ce(i, j)

    W = W0
    while W % 2 == 0 and W > 32:
        h = W // 2
        av = [v[:, :h] for v in vals]
        ai = [i_[:, :h] for i_ in idxs]
        bv = [v[:, h:] for v in vals]
        bi = [i_[:, h:] for i_ in idxs]
        for j in range(K_SZ):
            a, ia = av[j], ai[j]
            b, ib = bv[K_SZ - 1 - j], bi[K_SZ - 1 - j]
            c = lexlt(a, ia, b, ib)
            vals[j] = jnp.where(c, a, b)
            idxs[j] = jnp.where(c, ia, ib)
        for r in (8, 4, 2, 1):
            for j in range(K_SZ):
                if j & r == 0:
                    ce(j, j + r)
        W = h

    fv = jnp.concatenate(vals, axis=1)
    fi = jnp.concatenate(idxs, axis=1)
    cols = []
    for r in range(K_SZ):
        m = jnp.min(fv, axis=1, keepdims=True)
        am = jnp.min(jnp.where(fv == m, fi, jnp.int32(2**30)),
                     axis=1, keepdims=True)
        cols.append(am)
        if r != K_SZ - 1:
            fv = jnp.where((fv == m) & (fi == am), jnp.float32(jnp.inf), fv)
    idx_ref[...] = jnp.concatenate(cols, axis=1)


def _knn_call(qp8, sp8):
    return pl.pallas_call(
        _knn_body,
        grid=(Q_SZ // QB,),
        in_specs=[
            pl.BlockSpec((QB, 8), lambda i: (i, 0)),
            pl.BlockSpec((8, NP), lambda i: (0, 0)),
        ],
        out_specs=pl.BlockSpec((QB, K_SZ), lambda i: (i, 0)),
        out_shape=jax.ShapeDtypeStruct((Q_SZ, K_SZ), jnp.int32),
        scratch_shapes=[pltpu.VMEM((QB, NP), jnp.float32)],
    )(qp8, sp8)


def _make_gather():
    NC, NS = 2, 16
    NW = NC * NS
    B = Q_SZ * K_SZ
    b_per_w = B // NW
    CH = 128
    mesh = plsc.VectorSubcoreMesh(core_axis_name="c", subcore_axis_name="s")

    @functools.partial(
        pl.kernel,
        mesh=mesh,
        out_type=jax.ShapeDtypeStruct((B, D_SZ), jnp.float32),
        scratch_types=[
            pltpu.VMEM((CH,), jnp.int32),
            pltpu.VMEM((CH, D_SZ), jnp.float32),
            pltpu.SemaphoreType.DMA,
        ],
    )
    def gather(table_hbm, idx_hbm, out_hbm, idx_v, rows_v, sem):
        c = lax.axis_index("c")
        s = lax.axis_index("s")
        base = (s * NC + c) * b_per_w
        for j in range(b_per_w // CH):
            off = base + j * CH
            pltpu.sync_copy(idx_hbm.at[pl.ds(off, CH)], idx_v)
            pltpu.async_copy(table_hbm.at[idx_v], rows_v, sem).wait()
            pltpu.sync_copy(rows_v, out_hbm.at[pl.ds(off, CH)])

    return gather


def _epi_body(x_ref, feat_ref, wqT_ref, wkT_ref, wb_ref, wvT_ref, woT_ref,
              g_ref, b_ref, out_ref):
    x = x_ref[...]
    q = lax.dot_general(x, wqT_ref[...], (((1,), (0,)), ((), ())),
                        preferred_element_type=jnp.float32)
    logits = lax.dot_general(q, wkT_ref[...], (((1,), (0,)), ((), ())),
                             preferred_element_type=jnp.float32)
    logits = logits + wb_ref[0:1, :]
    mx = jnp.max(logits, axis=1, keepdims=True)
    e = jnp.exp(logits - mx)
    p = e / jnp.sum(e, axis=1, keepdims=True)
    f = feat_ref[...]
    agg = jnp.sum(f * p[:, :, None], axis=1)
    v = lax.dot_general(agg, wvT_ref[...], (((1,), (0,)), ((), ())),
                        preferred_element_type=jnp.float32)
    h = q * v
    o = lax.dot_general(h, woT_ref[...], (((1,), (0,)), ((), ())),
                        preferred_element_type=jnp.float32) + x
    mu = jnp.mean(o, axis=1, keepdims=True)
    var = jnp.mean((o - mu) ** 2, axis=1, keepdims=True)
    out_ref[...] = (o - mu) * lax.rsqrt(var + 1e-5) * g_ref[0:1, :] + b_ref[0:1, :]


def _epi_call(query, feat3, wqT, wkT, wb8, wvT, woT, g8, b8):
    return pl.pallas_call(
        _epi_body,
        grid=(Q_SZ // QBE,),
        in_specs=[
            pl.BlockSpec((QBE, D_SZ), lambda i: (i, 0)),
            pl.BlockSpec((QBE, K_SZ, D_SZ), lambda i: (i, 0, 0)),
            pl.BlockSpec((D_SZ, D_SZ), lambda i: (0, 0)),
            pl.BlockSpec((D_SZ, K_SZ), lambda i: (0, 0)),
            pl.BlockSpec((8, K_SZ), lambda i: (0, 0)),
            pl.BlockSpec((D_SZ, D_SZ), lambda i: (0, 0)),
            pl.BlockSpec((D_SZ, D_SZ), lambda i: (0, 0)),
            pl.BlockSpec((8, D_SZ), lambda i: (0, 0)),
            pl.BlockSpec((8, D_SZ), lambda i: (0, 0)),
        ],
        out_specs=pl.BlockSpec((QBE, D_SZ), lambda i: (i, 0)),
        out_shape=jax.ShapeDtypeStruct((Q_SZ, D_SZ), jnp.float32),
    )(query, feat3, wqT, wkT, wb8, wvT, woT, g8, b8)


def kernel(query, query_pos, inst_feats, sp_coords, w_q, w_v, w_o, w_k, w_b,
           ln_g, ln_b):
    qp8 = jnp.pad(query_pos, ((0, 0), (0, 5)))
    spT = jnp.pad(sp_coords.T, ((0, 0), (0, NP - N_SZ)),
                  constant_values=1.0e6)
    sp8 = jnp.pad(spT, ((0, 5), (0, 0)))

    idx = _knn_call(qp8, sp8)

    feat = _make_gather()(inst_feats, idx.reshape(-1))
    feat3 = feat.reshape(Q_SZ, K_SZ, D_SZ)

    wb8 = jnp.broadcast_to(w_b.reshape(1, K_SZ), (8, K_SZ))
    g8 = jnp.broadcast_to(ln_g.reshape(1, D_SZ), (8, D_SZ))
    b8 = jnp.broadcast_to(ln_b.reshape(1, D_SZ), (8, D_SZ))
    return _epi_call(query, feat3, w_q.T, w_k.T, wb8, w_v.T, w_o.T, g8, b8)
